# Initial kernel scaffold; baseline (speedup 1.0000x reference)
import jax
import jax.numpy as jnp
from jax.experimental import pallas as pl


def kernel(x, W1, b1, W2, b2):
    raise NotImplementedError("write your pallas kernel here")


# speedup vs baseline: 3.9138x; 3.9138x over previous
import functools

import jax
import jax.numpy as jnp
from jax.experimental import pallas as pl

T = 16384
D = 4096
E = 64
K = 8
RB = 1024


def _router_body(x_ref, w_ref, b_ref, eps_ref, probs_ref, idx_ref):
    acc = jnp.dot(x_ref[...], w_ref[...], preferred_element_type=jnp.float32)
    acc = acc + b_ref[...]
    logits = acc[:, :E]
    noise_logits = acc[:, E:]
    noisy = logits + eps_ref[...] * jax.nn.softplus(noise_logits)

    iota = jax.lax.broadcasted_iota(jnp.int32, (RB, E), 1)
    work = noisy
    sel = jnp.zeros((RB, E), jnp.bool_)
    idx_cols = []
    m0 = None
    for k in range(K):
        m = jnp.max(work, axis=1, keepdims=True)
        if k == 0:
            m0 = m
        a = jnp.min(jnp.where(work == m, iota, E), axis=1, keepdims=True)
        chosen = iota == a
        sel = jnp.logical_or(sel, chosen)
        work = jnp.where(chosen, -jnp.inf, work)
        idx_cols.append(a)

    unnorm = jnp.where(sel, jnp.exp(noisy - m0), 0.0)
    denom = jnp.sum(unnorm, axis=1, keepdims=True)
    probs_ref[...] = unnorm / denom
    idx_ref[...] = jnp.concatenate(idx_cols, axis=1)


@jax.jit
def kernel(x, W1, b1, W2, b2):
    eps = jax.random.normal(jax.random.key(42), (T, E), dtype=jnp.float32)
    Wc = jnp.concatenate([W1, W2], axis=1)
    bc = jnp.concatenate([b1, b2]).reshape(1, 2 * E)

    probs, indices = pl.pallas_call(
        _router_body,
        grid=(T // RB,),
        in_specs=[
            pl.BlockSpec((RB, D), lambda i: (i, 0)),
            pl.BlockSpec((D, 2 * E), lambda i: (0, 0)),
            pl.BlockSpec((1, 2 * E), lambda i: (0, 0)),
            pl.BlockSpec((RB, E), lambda i: (i, 0)),
        ],
        out_specs=[
            pl.BlockSpec((RB, E), lambda i: (i, 0)),
            pl.BlockSpec((RB, K), lambda i: (i, 0)),
        ],
        out_shape=[
            jax.ShapeDtypeStruct((T, E), jnp.float32),
            jax.ShapeDtypeStruct((T, K), jnp.int32),
        ],
    )(x, Wc, bc, eps)
    return probs, indices
